# Initial kernel scaffold; baseline (speedup 1.0000x reference)
import jax
import jax.numpy as jnp
from jax.experimental import pallas as pl


def kernel(input_, target):
    raise NotImplementedError("write your pallas kernel here")


# speedup vs baseline: 21.3448x; 21.3448x over previous
import math

import jax
import jax.numpy as jnp
from jax.experimental import pallas as pl
from jax.experimental.pallas import tpu as pltpu

DELTA_VAR = 0.5
DELTA_DIST = 2.0
ALPHA = 1.0
BETA = 1.0
GAMMA = 0.001
INSTANCE_W = 1.0
PMAPS_THRESHOLD = 0.9
TWO_SIGMA = DELTA_VAR * DELTA_VAR / -math.log(PMAPS_THRESHOLD)
C = 64
EPS = 1e-6

BLOCK = 4096

_DN_RHS_T = (((1,), (1,)), ((), ()))
_DN_MATMUL = (((1,), (0,)), ((), ()))


def _dot(a, b, dn):
    return jax.lax.dot_general(
        a, b, dn,
        preferred_element_type=jnp.float32,
        precision=jax.lax.Precision.HIGHEST)


def _loss_kernel(p_total,
                 emb_ref, tgt_ref, out_ref,
                 sums_ref, counts_ref, means_ref, mun2_ref, invc_ref,
                 var_ref, inter_ref, p2_ref):
    p_id = pl.program_id(0)
    i_id = pl.program_id(1)
    nblocks = pl.num_programs(1)

    e = emb_ref[...]
    t = tgt_ref[...]
    b = e.shape[1]
    ids = jax.lax.broadcasted_iota(jnp.int32, (C, b), 0)
    ohf = (ids == t).astype(jnp.float32)

    @pl.when(jnp.logical_and(p_id == 0, i_id == 0))
    def _init():
        sums_ref[...] = jnp.zeros_like(sums_ref)
        counts_ref[...] = jnp.zeros_like(counts_ref)
        var_ref[0, 0] = 0.0
        inter_ref[0, 0] = 0.0
        p2_ref[0, 0] = 0.0

    @pl.when(p_id == 0)
    def _pass0():
        sums_ref[...] += _dot(ohf, e, _DN_RHS_T)
        counts_ref[...] += jnp.sum(ohf, axis=1, keepdims=True)

    @pl.when(jnp.logical_and(p_id == 1, i_id == 0))
    def _means():
        safe = jnp.maximum(counts_ref[...], 1.0)
        m = sums_ref[...] / safe
        means_ref[...] = m
        mun2_ref[...] = jnp.sum(m * m, axis=1, keepdims=True)
        invc_ref[...] = 1.0 / safe

    @pl.when(p_id == 1)
    def _pass1():
        means = means_ref[...]
        mun2 = mun2_ref[...]
        g = _dot(means, e, _DN_MATMUL)
        en2 = jnp.sum(e * e, axis=0, keepdims=True)
        d2 = jnp.maximum(en2 - 2.0 * g + mun2, 0.0)
        d2sel = jnp.sum(ohf * d2, axis=0, keepdims=True)
        hinge = jnp.maximum(jnp.sqrt(d2sel) - DELTA_VAR, 0.0) ** 2
        w = jnp.sum(ohf * invc_ref[...], axis=0, keepdims=True)
        var_ref[0, 0] += jnp.sum(hinge * w)
        pm = jnp.exp(-d2 / TWO_SIGMA)
        pm = jnp.where(ids == 0, 0.0, pm)
        inter_ref[0, 0] += jnp.sum(ohf * pm)
        p2_ref[0, 0] += jnp.sum(pm * pm)

    @pl.when(jnp.logical_and(p_id == 1, i_id == nblocks - 1))
    def _final():
        means = means_ref[...]
        mun2 = mun2_ref[...]
        gm = _dot(means, means, _DN_RHS_T)
        ri = jax.lax.broadcasted_iota(jnp.int32, (C, C), 0)
        ci = jax.lax.broadcasted_iota(jnp.int32, (C, C), 1)
        diag = jnp.where(ri == ci, gm, 0.0)
        mun2_row = jnp.sum(diag, axis=0, keepdims=True)
        dd2 = jnp.maximum(mun2 + mun2_row - 2.0 * gm, 0.0)
        dmat = jnp.sqrt(dd2)
        hinged = jnp.where(
            ri == ci, 0.0,
            jnp.maximum(2.0 * DELTA_DIST - dmat, 0.0) ** 2)
        distance_term = jnp.sum(hinged) / (C * (C - 1))
        variance_term = var_ref[0, 0] / C
        reg_term = jnp.sum(jnp.sqrt(mun2)) / C
        rows = jax.lax.broadcasted_iota(jnp.int32, (C, 1), 0)
        count0 = jnp.sum(jnp.where(rows == 0, counts_ref[...], 0.0))
        m2 = p_total - count0
        denom = jnp.maximum(p2_ref[0, 0] + m2, EPS)
        dice = 2.0 * inter_ref[0, 0] / denom
        instance_term = 1.0 - dice
        loss = (ALPHA * variance_term + BETA * distance_term
                + GAMMA * reg_term + INSTANCE_W * instance_term)
        out_ref[0, 0] = 2.0 * loss


@jax.jit
def _run(emb, tgt):
    p = emb.shape[1]
    nb = p // BLOCK
    import functools
    out = pl.pallas_call(
        functools.partial(_loss_kernel, float(p)),
        grid=(2, nb),
        in_specs=[
            pl.BlockSpec((16, BLOCK), lambda pp, i: (0, i)),
            pl.BlockSpec((1, BLOCK), lambda pp, i: (0, i)),
        ],
        out_specs=pl.BlockSpec((1, 1), lambda pp, i: (0, 0),
                               memory_space=pltpu.SMEM),
        out_shape=jax.ShapeDtypeStruct((1, 1), jnp.float32),
        scratch_shapes=[
            pltpu.VMEM((C, 16), jnp.float32),
            pltpu.VMEM((C, 1), jnp.float32),
            pltpu.VMEM((C, 16), jnp.float32),
            pltpu.VMEM((C, 1), jnp.float32),
            pltpu.VMEM((C, 1), jnp.float32),
            pltpu.SMEM((1, 1), jnp.float32),
            pltpu.SMEM((1, 1), jnp.float32),
            pltpu.SMEM((1, 1), jnp.float32),
        ],
    )(emb, tgt)
    return out[0, 0]


def kernel(input_, target):
    n_batches = input_.shape[0]
    emb = input_[n_batches - 1].reshape(16, -1)
    tgt = target[n_batches - 1, 0].reshape(1, -1)
    return _run(emb, tgt) / n_batches


# speedup vs baseline: 28.3747x; 1.3293x over previous
import math

import jax
import jax.numpy as jnp
from jax.experimental import pallas as pl
from jax.experimental.pallas import tpu as pltpu

DELTA_VAR = 0.5
DELTA_DIST = 2.0
ALPHA = 1.0
BETA = 1.0
GAMMA = 0.001
INSTANCE_W = 1.0
PMAPS_THRESHOLD = 0.9
TWO_SIGMA = DELTA_VAR * DELTA_VAR / -math.log(PMAPS_THRESHOLD)
C = 64
EPS = 1e-6

BLOCK = 4096

_DN_RHS_T = (((1,), (1,)), ((), ()))
_DN_MATMUL = (((1,), (0,)), ((), ()))


def _dot(a, b, dn):
    return jax.lax.dot_general(
        a, b, dn,
        preferred_element_type=jnp.float32,
        precision=jax.lax.Precision.DEFAULT)


def _loss_kernel(p_total,
                 emb_ref, tgt_ref, out_ref,
                 sums_ref, counts_ref, means_ref, mun2_ref, invc_ref,
                 var_ref, inter_ref, p2_ref):
    p_id = pl.program_id(0)
    i_id = pl.program_id(1)
    nblocks = pl.num_programs(1)

    e = emb_ref[...]
    t = tgt_ref[...]
    b = e.shape[1]
    ids = jax.lax.broadcasted_iota(jnp.int32, (C, b), 0)
    ohf = (ids == t).astype(jnp.float32)

    @pl.when(jnp.logical_and(p_id == 0, i_id == 0))
    def _init():
        sums_ref[...] = jnp.zeros_like(sums_ref)
        counts_ref[...] = jnp.zeros_like(counts_ref)
        var_ref[0, 0] = 0.0
        inter_ref[0, 0] = 0.0
        p2_ref[0, 0] = 0.0

    @pl.when(p_id == 0)
    def _pass0():
        sums_ref[...] += _dot(ohf, e, _DN_RHS_T)
        counts_ref[...] += jnp.sum(ohf, axis=1, keepdims=True)

    @pl.when(jnp.logical_and(p_id == 1, i_id == 0))
    def _means():
        safe = jnp.maximum(counts_ref[...], 1.0)
        m = sums_ref[...] / safe
        means_ref[...] = m
        mun2_ref[...] = jnp.sum(m * m, axis=1, keepdims=True)
        invc_ref[...] = 1.0 / safe

    @pl.when(p_id == 1)
    def _pass1():
        means = means_ref[...]
        mun2 = mun2_ref[...]
        g = _dot(means, e, _DN_MATMUL)
        en2 = jnp.sum(e * e, axis=0, keepdims=True)
        d2 = jnp.maximum(en2 - 2.0 * g + mun2, 0.0)
        d2sel = jnp.sum(ohf * d2, axis=0, keepdims=True)
        hinge = jnp.maximum(jnp.sqrt(d2sel) - DELTA_VAR, 0.0) ** 2
        w = jnp.sum(ohf * invc_ref[...], axis=0, keepdims=True)
        var_ref[0, 0] += jnp.sum(hinge * w)
        pm = jnp.exp(-d2 / TWO_SIGMA)
        pm = jnp.where(ids == 0, 0.0, pm)
        inter_ref[0, 0] += jnp.sum(ohf * pm)
        p2_ref[0, 0] += jnp.sum(pm * pm)

    @pl.when(jnp.logical_and(p_id == 1, i_id == nblocks - 1))
    def _final():
        means = means_ref[...]
        mun2 = mun2_ref[...]
        gm = _dot(means, means, _DN_RHS_T)
        ri = jax.lax.broadcasted_iota(jnp.int32, (C, C), 0)
        ci = jax.lax.broadcasted_iota(jnp.int32, (C, C), 1)
        diag = jnp.where(ri == ci, gm, 0.0)
        mun2_row = jnp.sum(diag, axis=0, keepdims=True)
        dd2 = jnp.maximum(mun2 + mun2_row - 2.0 * gm, 0.0)
        dmat = jnp.sqrt(dd2)
        hinged = jnp.where(
            ri == ci, 0.0,
            jnp.maximum(2.0 * DELTA_DIST - dmat, 0.0) ** 2)
        distance_term = jnp.sum(hinged) / (C * (C - 1))
        variance_term = var_ref[0, 0] / C
        reg_term = jnp.sum(jnp.sqrt(mun2)) / C
        rows = jax.lax.broadcasted_iota(jnp.int32, (C, 1), 0)
        count0 = jnp.sum(jnp.where(rows == 0, counts_ref[...], 0.0))
        m2 = p_total - count0
        denom = jnp.maximum(p2_ref[0, 0] + m2, EPS)
        dice = 2.0 * inter_ref[0, 0] / denom
        instance_term = 1.0 - dice
        loss = (ALPHA * variance_term + BETA * distance_term
                + GAMMA * reg_term + INSTANCE_W * instance_term)
        out_ref[0, 0] = 2.0 * loss


@jax.jit
def _run(emb, tgt):
    p = emb.shape[1]
    nb = p // BLOCK
    import functools
    out = pl.pallas_call(
        functools.partial(_loss_kernel, float(p)),
        grid=(2, nb),
        in_specs=[
            pl.BlockSpec((16, BLOCK), lambda pp, i: (0, i)),
            pl.BlockSpec((1, BLOCK), lambda pp, i: (0, i)),
        ],
        out_specs=pl.BlockSpec((1, 1), lambda pp, i: (0, 0),
                               memory_space=pltpu.SMEM),
        out_shape=jax.ShapeDtypeStruct((1, 1), jnp.float32),
        scratch_shapes=[
            pltpu.VMEM((C, 16), jnp.float32),
            pltpu.VMEM((C, 1), jnp.float32),
            pltpu.VMEM((C, 16), jnp.float32),
            pltpu.VMEM((C, 1), jnp.float32),
            pltpu.VMEM((C, 1), jnp.float32),
            pltpu.SMEM((1, 1), jnp.float32),
            pltpu.SMEM((1, 1), jnp.float32),
            pltpu.SMEM((1, 1), jnp.float32),
        ],
    )(emb, tgt)
    return out[0, 0]


def kernel(input_, target):
    n_batches = input_.shape[0]
    emb = input_[n_batches - 1].reshape(16, -1)
    tgt = target[n_batches - 1, 0].reshape(1, -1)
    return _run(emb, tgt) / n_batches
